# Initial kernel scaffold; baseline (speedup 1.0000x reference)
import jax
import jax.numpy as jnp
from jax.experimental import pallas as pl


def kernel(x, W1, b1, W2, b2):
    raise NotImplementedError("write your pallas kernel here")


# speedup vs baseline: 1.2139x; 1.2139x over previous
import numpy as np
import jax
import jax.numpy as jnp
from jax.experimental import pallas as pl
from jax.experimental.pallas import tpu as pltpu

_WIN = 6


def _dct_colsum():
    n = np.arange(_WIN)
    D = np.cos(np.pi * (n[None, :] + 0.5) * n[:, None] / _WIN)
    scale = np.full(_WIN, np.sqrt(2.0 / _WIN))
    scale[0] = np.sqrt(1.0 / _WIN)
    D = D * scale[:, None]
    return tuple(float(v) for v in D.sum(axis=0))


_TSUM = _dct_colsum()


def kernel(x, W1, b1, W2, b2):
    B, L, C = x.shape
    S = L // _WIN
    H = W1.shape[2]
    P = W2.shape[2]
    CB = 128
    n_cb = (C + CB - 1) // CB
    Bb = 16
    nbh = B // (2 * Bb)

    x4 = x.reshape(B, S, _WIN, C)
    W1t = W1.transpose(1, 2, 0)
    W2t = W2.transpose(1, 2, 0)
    b1t = b1.T
    b2t = b2.T

    def body(x_ref, w1_ref, b1_ref, w2_ref, b2_ref, o_ref, w1bc, w2bc, bt_ref):
        bi = pl.program_id(2)

        @pl.when(bi == 0)
        def _():
            w1 = w1_ref[...]
            w2 = w2_ref[...]
            w1bc[...] = jnp.broadcast_to(w1[:, :, None, :], (S, H, _WIN, CB))
            w2bc[...] = jnp.broadcast_to(w2[:, :, None, :], (H, P, _WIN, CB))
            beta = b2_ref[...]
            for h in range(H):
                beta = beta + b1_ref[h:h + 1, :] * w2[h]
            for n in range(_WIN):
                bt_ref[:, n, :] = beta * _TSUM[n]

        def per_b(b, carry):
            xb = x_ref[b]
            mean_b = jnp.mean(xb, axis=(0, 1), keepdims=True)
            xc = xb - mean_b
            hacc = None
            for s in range(S):
                term = xc[s][None] * w1bc[s]
                hacc = term if hacc is None else hacc + term
            acc = bt_ref[...]
            for h in range(H):
                acc = acc + hacc[h][None] * w2bc[h]
            o_ref[b] = acc + mean_b
            return carry

        jax.lax.fori_loop(0, Bb, per_b, 0)

    out4 = pl.pallas_call(
        body,
        out_shape=jax.ShapeDtypeStruct((B, S, _WIN, C), jnp.float32),
        grid=(2, n_cb, nbh),
        in_specs=[
            pl.BlockSpec((Bb, S, _WIN, CB), lambda o, c, bi: (o * nbh + bi, 0, 0, c)),
            pl.BlockSpec((S, H, CB), lambda o, c, bi: (0, 0, c)),
            pl.BlockSpec((H, CB), lambda o, c, bi: (0, c)),
            pl.BlockSpec((H, P, CB), lambda o, c, bi: (0, 0, c)),
            pl.BlockSpec((P, CB), lambda o, c, bi: (0, c)),
        ],
        out_specs=pl.BlockSpec((Bb, S, _WIN, CB), lambda o, c, bi: (o * nbh + bi, 0, 0, c)),
        scratch_shapes=[
            pltpu.VMEM((S, H, _WIN, CB), jnp.float32),
            pltpu.VMEM((H, P, _WIN, CB), jnp.float32),
            pltpu.VMEM((P, _WIN, CB), jnp.float32),
        ],
        compiler_params=pltpu.CompilerParams(
            dimension_semantics=("parallel", "arbitrary", "arbitrary"),
        ),
        name="esn_ltf_fused",
    )(x4, W1t, b1t, W2t, b2t)

    return out4.reshape(B, L, C)
